# baseline (speedup 1.0000x reference)
import functools

import jax
import jax.numpy as jnp
from jax import lax
from jax.experimental import pallas as pl
from jax.experimental.pallas import tpu as pltpu
from jax.experimental.pallas import tpu_sc as plsc

N_NODES = 10000
D = 128
E = 320000
NC = 2
NS = 16
L = 16
NW = NC * NS
E_PER_W = E // NW
CH = 80
N_CH = E_PER_W // CH
G_PER_CH = CH // L

_mesh = plsc.VectorSubcoreMesh(core_axis_name="c", subcore_axis_name="s")


@functools.partial(
    pl.kernel,
    out_type=jax.ShapeDtypeStruct((E,), jnp.float32),
    mesh=_mesh,
    scratch_types=[
        pltpu.VMEM((E_PER_W,), jnp.int32),
        pltpu.VMEM((E_PER_W,), jnp.int32),
        pltpu.VMEM((CH, D), jnp.float32),
        pltpu.VMEM((CH, D), jnp.float32),
        pltpu.VMEM((CH, D), jnp.float32),
        pltpu.VMEM((CH, D), jnp.float32),
        pltpu.VMEM((E_PER_W,), jnp.float32),
        pltpu.SemaphoreType.DMA,
        pltpu.SemaphoreType.DMA,
        pltpu.SemaphoreType.DMA,
        pltpu.SemaphoreType.DMA,
    ],
    compiler_params=pltpu.CompilerParams(needs_layout_passes=False),
)
def _score_kernel(h_hbm, src_hbm, dst_hbm, out_hbm,
                  idx_s, idx_d, rows_s0, rows_s1, rows_d0, rows_d1, out_v,
                  sem_s0, sem_s1, sem_d0, sem_d1):
    wid = lax.axis_index("s") * NC + lax.axis_index("c")
    wbase = pl.multiple_of(wid * E_PER_W, 8)
    iota = lax.iota(jnp.int32, L)

    pltpu.sync_copy(src_hbm.at[pl.ds(wbase, E_PER_W)], idx_s)
    pltpu.sync_copy(dst_hbm.at[pl.ds(wbase, E_PER_W)], idx_d)

    rows = ((rows_s0, rows_d0, sem_s0, sem_d0),
            (rows_s1, rows_d1, sem_s1, sem_d1))

    def fire(c, buf):
        rs, rd, ss, sd = rows[buf]
        off = pl.multiple_of(c * CH, 8)
        pltpu.async_copy(h_hbm.at[idx_s.at[pl.ds(off, CH)]], rs, ss)
        pltpu.async_copy(h_hbm.at[idx_d.at[pl.ds(off, CH)]], rd, sd)

    def drain(buf):
        rs, rd, ss, sd = rows[buf]
        pltpu.make_async_copy(h_hbm.at[idx_s.at[pl.ds(0, CH)]], rs, ss).wait()
        pltpu.make_async_copy(h_hbm.at[idx_d.at[pl.ds(0, CH)]], rd, sd).wait()

    def compute(c, buf):
        rs, rd, _, _ = rows[buf]

        @pl.loop(0, G_PER_CH)
        def _group(g):
            edge = g * L + iota
            accs = [jnp.zeros((L,), jnp.float32) for _ in range(4)]
            for dcol in range(D):
                colv = (iota + dcol) & (D - 1)
                a = plsc.load_gather(rs, [edge, colv])
                b = plsc.load_gather(rd, [edge, colv])
                accs[dcol % 4] = accs[dcol % 4] + a * b
            acc = (accs[0] + accs[1]) + (accs[2] + accs[3])
            off = pl.multiple_of(c * CH + g * L, 8)
            out_v[pl.ds(off, L)] = acc

    fire(0, 0)

    @pl.loop(0, N_CH - 1, step=2)
    def _chunk(c):
        fire(c + 1, 1)
        drain(0)
        compute(c, 0)
        fire(c + 2, 0)
        drain(1)
        compute(c + 1, 1)

    drain(0)
    compute(N_CH - 1, 0)

    pltpu.sync_copy(out_v, out_hbm.at[pl.ds(wbase, E_PER_W)])


def kernel(u_f, v_f, edge_index):
    h = jnp.concatenate([u_f, v_f], axis=0)
    ei = edge_index.astype(jnp.int32)
    score = _score_kernel(h, ei[0], ei[1])
    return score.reshape(E, 1)


# speedup vs baseline: 2.9864x; 2.9864x over previous
import functools

import jax
import jax.numpy as jnp
from jax import lax
from jax.experimental import pallas as pl
from jax.experimental.pallas import tpu as pltpu
from jax.experimental.pallas import tpu_sc as plsc

N_NODES = 10000
D = 128
E = 320000
NC = 2
NS = 16
L = 16
NW = NC * NS
E_PER_W = E // NW
CH = 80
N_CH = E_PER_W // CH
G_PER_CH = CH // L

_mesh = plsc.VectorSubcoreMesh(core_axis_name="c", subcore_axis_name="s")


@functools.partial(
    pl.kernel,
    out_type=jax.ShapeDtypeStruct((E,), jnp.float32),
    mesh=_mesh,
    scratch_types=[
        pltpu.VMEM((E_PER_W,), jnp.int32),
        pltpu.VMEM((E_PER_W,), jnp.int32),
        pltpu.VMEM((CH, D), jnp.float32),
        pltpu.VMEM((CH, D), jnp.float32),
        pltpu.VMEM((CH, D), jnp.float32),
        pltpu.VMEM((CH, D), jnp.float32),
        pltpu.VMEM((E_PER_W,), jnp.float32),
        pltpu.SemaphoreType.DMA,
        pltpu.SemaphoreType.DMA,
        pltpu.SemaphoreType.DMA,
        pltpu.SemaphoreType.DMA,
    ],
    compiler_params=pltpu.CompilerParams(needs_layout_passes=False),
)
def _score_kernel(h_hbm, src_hbm, dst_hbm, out_hbm,
                  idx_s, idx_d, rows_s0, rows_s1, rows_d0, rows_d1, out_v,
                  sem_s0, sem_s1, sem_d0, sem_d1):
    wid = lax.axis_index("s") * NC + lax.axis_index("c")
    wbase = pl.multiple_of(wid * E_PER_W, 8)
    iota = lax.iota(jnp.int32, L)

    pltpu.sync_copy(src_hbm.at[pl.ds(wbase, E_PER_W)], idx_s)
    pltpu.sync_copy(dst_hbm.at[pl.ds(wbase, E_PER_W)], idx_d)

    rows = ((rows_s0, rows_d0, sem_s0, sem_d0),
            (rows_s1, rows_d1, sem_s1, sem_d1))

    def fire(c, buf):
        rs, rd, ss, sd = rows[buf]
        off = pl.multiple_of(c * CH, 8)
        pltpu.async_copy(h_hbm.at[idx_s.at[pl.ds(off, CH)]], rs, ss)
        pltpu.async_copy(h_hbm.at[idx_d.at[pl.ds(off, CH)]], rd, sd)

    def drain(buf):
        rs, rd, ss, sd = rows[buf]
        pltpu.make_async_copy(h_hbm.at[idx_s.at[pl.ds(0, CH)]], rs, ss).wait()
        pltpu.make_async_copy(h_hbm.at[idx_d.at[pl.ds(0, CH)]], rd, sd).wait()

    def compute(c, buf):
        rs, rd, _, _ = rows[buf]

        @pl.loop(0, G_PER_CH)
        def _group(g):
            edge = g * L + iota

            init = (iota, jnp.zeros((L, ), jnp.float32),
                    jnp.zeros((L, ), jnp.float32),
                    jnp.zeros((L, ), jnp.float32),
                    jnp.zeros((L, ), jnp.float32))

            @pl.loop(0, D, init_carry=init, unroll=8)
            def _col(dcol, carry):
                colv, a0, a1, a2, a3 = carry
                a = plsc.load_gather(rs, [edge, colv])
                b = plsc.load_gather(rd, [edge, colv])
                return ((colv + 1) & (D - 1), a1, a2, a3, a0 + a * b)

            _, a0, a1, a2, a3 = _col
            acc = (a0 + a1) + (a2 + a3)
            off = pl.multiple_of(c * CH + g * L, 8)
            out_v[pl.ds(off, L)] = acc

    fire(0, 0)

    @pl.loop(0, N_CH - 1, step=2)
    def _chunk(c):
        fire(c + 1, 1)
        drain(0)
        compute(c, 0)
        fire(c + 2, 0)
        drain(1)
        compute(c + 1, 1)

    drain(0)
    compute(N_CH - 1, 0)

    pltpu.sync_copy(out_v, out_hbm.at[pl.ds(wbase, E_PER_W)])


def kernel(u_f, v_f, edge_index):
    h = jnp.concatenate([u_f, v_f], axis=0)
    ei = edge_index.astype(jnp.int32)
    score = _score_kernel(h, ei[0], ei[1])
    return score.reshape(E, 1)


# speedup vs baseline: 3.4495x; 1.1551x over previous
import functools

import jax
import jax.numpy as jnp
from jax import lax
from jax.experimental import pallas as pl
from jax.experimental.pallas import tpu as pltpu
from jax.experimental.pallas import tpu_sc as plsc

N_NODES = 10000
D = 128
E = 320000
NC = 2
NS = 16
L = 16
NW = NC * NS
E_PER_W = E // NW
CH = 80
N_CH = E_PER_W // CH
G_PER_CH = CH // L
NBUF = 4

_mesh = plsc.VectorSubcoreMesh(core_axis_name="c", subcore_axis_name="s")


@functools.partial(
    pl.kernel,
    out_type=jax.ShapeDtypeStruct((E,), jnp.float32),
    mesh=_mesh,
    scratch_types=[
        pltpu.VMEM((E_PER_W,), jnp.int32),
        pltpu.VMEM((E_PER_W,), jnp.int32),
        [pltpu.VMEM((CH, D), jnp.float32) for _ in range(NBUF)],
        [pltpu.VMEM((CH, D), jnp.float32) for _ in range(NBUF)],
        pltpu.VMEM((E_PER_W,), jnp.float32),
        [pltpu.SemaphoreType.DMA for _ in range(NBUF)],
        [pltpu.SemaphoreType.DMA for _ in range(NBUF)],
    ],
    compiler_params=pltpu.CompilerParams(needs_layout_passes=False),
)
def _score_kernel(h_hbm, src_hbm, dst_hbm, out_hbm,
                  idx_s, idx_d, rows_s, rows_d, out_v, sems_s, sems_d):
    wid = lax.axis_index("s") * NC + lax.axis_index("c")
    wbase = pl.multiple_of(wid * E_PER_W, 8)
    iota = lax.iota(jnp.int32, L)

    pltpu.sync_copy(src_hbm.at[pl.ds(wbase, E_PER_W)], idx_s)
    pltpu.sync_copy(dst_hbm.at[pl.ds(wbase, E_PER_W)], idx_d)

    def fire(c, buf):
        off = pl.multiple_of(c * CH, 8)
        pltpu.async_copy(h_hbm.at[idx_s.at[pl.ds(off, CH)]], rows_s[buf],
                         sems_s[buf])
        pltpu.async_copy(h_hbm.at[idx_d.at[pl.ds(off, CH)]], rows_d[buf],
                         sems_d[buf])

    def drain(buf):
        pltpu.make_async_copy(h_hbm.at[idx_s.at[pl.ds(0, CH)]], rows_s[buf],
                              sems_s[buf]).wait()
        pltpu.make_async_copy(h_hbm.at[idx_d.at[pl.ds(0, CH)]], rows_d[buf],
                              sems_d[buf]).wait()

    def compute(c, buf):
        rs = rows_s[buf]
        rd = rows_d[buf]

        @pl.loop(0, G_PER_CH)
        def _group(g):
            edge = g * L + iota

            init = (iota, jnp.zeros((L,), jnp.float32),
                    jnp.zeros((L,), jnp.float32),
                    jnp.zeros((L,), jnp.float32),
                    jnp.zeros((L,), jnp.float32))

            @pl.loop(0, D, init_carry=init, unroll=16)
            def _col(dcol, carry):
                colv, a0, a1, a2, a3 = carry
                a = plsc.load_gather(rs, [edge, colv])
                b = plsc.load_gather(rd, [edge, colv])
                return ((colv + 1) & (D - 1), a1, a2, a3, a0 + a * b)

            _, a0, a1, a2, a3 = _col
            acc = (a0 + a1) + (a2 + a3)
            off = pl.multiple_of(c * CH + g * L, 8)
            out_v[pl.ds(off, L)] = acc

    for b in range(NBUF - 1):
        fire(b, b)

    @pl.loop(0, N_CH - 1, step=NBUF)
    def _chunk(c):
        for k in range(NBUF):
            nxt = c + k + NBUF - 1

            @pl.when(nxt < N_CH)
            def _():
                fire(nxt, (k + NBUF - 1) % NBUF)

            drain(k)
            compute(c + k, k)

    drain((N_CH - 1) % NBUF)
    compute(N_CH - 1, (N_CH - 1) % NBUF)

    pltpu.sync_copy(out_v, out_hbm.at[pl.ds(wbase, E_PER_W)])


def kernel(u_f, v_f, edge_index):
    h = jnp.concatenate([u_f, v_f], axis=0)
    ei = edge_index.astype(jnp.int32)
    score = _score_kernel(h, ei[0], ei[1])
    return score.reshape(E, 1)
